# Initial kernel scaffold; baseline (speedup 1.0000x reference)
import jax
import jax.numpy as jnp
from jax.experimental import pallas as pl


def kernel(V, E, edges, W_edge, W_att, b_att):
    raise NotImplementedError("write your pallas kernel here")


# speedup vs baseline: 1968.8658x; 1968.8658x over previous
import functools

import jax
import jax.numpy as jnp
from jax import lax
from jax.experimental import pallas as pl
from jax.experimental.pallas import tpu as pltpu
from jax.experimental.pallas import tpu_sc as plsc

N_NODES = 10000
NP = 10240
N_EDGES = 160000
D = 128
NH = 4
OPH = 32

NC = 2
NS = 16
NW = NC * NS
EW = N_EDGES // NW
CHUNK = 1000
NCHUNK = EW // CHUNK
NGROUP = -(-CHUNK // 16)

_NEG = -1e30


def _node_body(v_ref, wcat_ref, wsr_ref, bvec_ref, h_ref, a_ref):
    h = jnp.dot(v_ref[...], wcat_ref[...], preferred_element_type=jnp.float32)
    h_ref[...] = h
    a_ref[...] = (
        jnp.dot(h, wsr_ref[...], preferred_element_type=jnp.float32)
        + bvec_ref[...]
    )


def _node_tables(V2p, wcat, wsr, bvec):
    blk = 2048
    grid = NP // blk
    return pl.pallas_call(
        _node_body,
        grid=(grid,),
        in_specs=[
            pl.BlockSpec((blk, D), lambda i: (i, 0)),
            pl.BlockSpec((D, D), lambda i: (0, 0)),
            pl.BlockSpec((D, 8), lambda i: (0, 0)),
            pl.BlockSpec((1, 8), lambda i: (0, 0)),
        ],
        out_specs=[
            pl.BlockSpec((blk, D), lambda i: (i, 0)),
            pl.BlockSpec((blk, 8), lambda i: (i, 0)),
        ],
        out_shape=[
            jax.ShapeDtypeStruct((NP, D), jnp.float32),
            jax.ShapeDtypeStruct((NP, 8), jnp.float32),
        ],
    )(V2p, wcat, wsr, bvec)


def _escore_body(waep_ref, e_ref, o0_ref, o1_ref, o2_ref, o3_ref):
    r = lax.dot_general(
        waep_ref[...], e_ref[...],
        (((1,), (1,)), ((), ())),
        preferred_element_type=jnp.float32,
    )
    for h, o_ref in enumerate((o0_ref, o1_ref, o2_ref, o3_ref)):
        o_ref[...] = r[h]


def _edge_scores(E2, waep):
    blk = 4096
    grid = pl.cdiv(N_EDGES, blk)
    return pl.pallas_call(
        _escore_body,
        grid=(grid,),
        in_specs=[
            pl.BlockSpec((8, D), lambda i: (0, 0)),
            pl.BlockSpec((blk, D), lambda i: (i, 0)),
        ],
        out_specs=[pl.BlockSpec((blk,), lambda i: (i,))] * NH,
        out_shape=[jax.ShapeDtypeStruct((N_EDGES,), jnp.float32)] * NH,
    )(waep, E2)


def _logits_body(a_hbm, e0_hbm, e1_hbm, e2_hbm, e3_hbm, src_hbm, dst_hbm,
                 ap0_hbm, ap1_hbm, ap2_hbm, ap3_hbm, m_hbm,
                 a_v, e_v0, e_v1, e_v2, e_v3, src_v, dst_v, stage_v):
    e_hbms = (e0_hbm, e1_hbm, e2_hbm, e3_hbm)
    ap_hbms = (ap0_hbm, ap1_hbm, ap2_hbm, ap3_hbm)
    e_vs = (e_v0, e_v1, e_v2, e_v3)
    cid = lax.axis_index("c")
    sid = lax.axis_index("s")
    wid = sid * NC + cid
    iota = lax.iota(jnp.int32, 16)

    pltpu.sync_copy(a_hbm, a_v)
    base_w = wid * EW

    vmaxs = (jnp.full((16,), _NEG),) * NH
    for c in range(NCHUNK):
        base = base_w + c * CHUNK
        pltpu.sync_copy(src_hbm.at[pl.ds(base, CHUNK)],
                        src_v.at[pl.ds(0, CHUNK)])
        pltpu.sync_copy(dst_hbm.at[pl.ds(base, CHUNK)],
                        dst_v.at[pl.ds(0, CHUNK)])
        for h in range(NH):
            pltpu.sync_copy(e_hbms[h].at[pl.ds(base, CHUNK)],
                            e_vs[h].at[pl.ds(0, CHUNK)])

        def group(g, carry):
            off = g * 16
            mask = iota < (CHUNK - off)
            src16 = src_v[pl.ds(off, 16)]
            dst16 = dst_v[pl.ds(off, 16)]
            src16 = jnp.minimum(jnp.maximum(src16, 0), N_NODES - 1)
            dst16 = jnp.minimum(jnp.maximum(dst16, 0), N_NODES - 1)
            out = []
            for h in range(NH):
                a_s = plsc.load_gather(a_v, [src16 * 8 + h])
                a_r = plsc.load_gather(a_v, [dst16 * 8 + (NH + h)])
                x = a_s + a_r + e_vs[h][pl.ds(off, 16)]
                att = jnp.where(x >= 0, x, x * jnp.float32(0.2))
                att = jnp.where(mask, att, _NEG)
                e_vs[h][pl.ds(off, 16)] = att
                out.append(jnp.maximum(carry[h], att))
            return tuple(out)

        vmaxs = lax.fori_loop(0, NGROUP, group, vmaxs)
        for h in range(NH):
            pltpu.sync_copy(e_vs[h].at[pl.ds(0, CHUNK)],
                            ap_hbms[h].at[pl.ds(base, CHUNK)])

    comb = jnp.full((16,), _NEG)
    for h in range(NH):
        comb = jnp.where(iota == h, jnp.max(vmaxs[h]), comb)
    stage_v[...] = comb
    pltpu.sync_copy(stage_v, m_hbm.at[pl.ds(wid * 16, 16)])


def _edge_logits(a_tab, e_sc, src, dst):
    mesh = plsc.VectorSubcoreMesh(core_axis_name="c", subcore_axis_name="s")
    kern = functools.partial(
        pl.kernel,
        out_type=[jax.ShapeDtypeStruct((N_EDGES,), jnp.float32)] * NH
        + [jax.ShapeDtypeStruct((NW * 16,), jnp.float32)],
        mesh=mesh,
        scratch_types=[
            pltpu.VMEM((NP * 8,), jnp.float32),
            pltpu.VMEM((1008,), jnp.float32),
            pltpu.VMEM((1008,), jnp.float32),
            pltpu.VMEM((1008,), jnp.float32),
            pltpu.VMEM((1008,), jnp.float32),
            pltpu.VMEM((1008,), jnp.int32),
            pltpu.VMEM((1008,), jnp.int32),
            pltpu.VMEM((16,), jnp.float32),
        ],
        compiler_params=pltpu.CompilerParams(needs_layout_passes=False),
    )(_logits_body)
    return kern(a_tab, e_sc[0], e_sc[1], e_sc[2], e_sc[3], src, dst)


def _segsum_body(ap0_hbm, ap1_hbm, ap2_hbm, ap3_hbm, src_hbm, m_hbm, s_hbm,
                 s_v, ap_v0, ap_v1, ap_v2, ap_v3, src_v, m_v):
    ap_hbms = (ap0_hbm, ap1_hbm, ap2_hbm, ap3_hbm)
    ap_vs = (ap_v0, ap_v1, ap_v2, ap_v3)
    cid = lax.axis_index("c")
    sid = lax.axis_index("s")
    wid = sid * NC + cid
    iota = lax.iota(jnp.int32, 16)

    pltpu.sync_copy(m_hbm, m_v)
    m_sc = []
    for h in range(NH):
        g1 = plsc.load_gather(m_v, [iota * 16 + h])
        g2 = plsc.load_gather(m_v, [(iota + 16) * 16 + h])
        m_sc.append(jnp.max(jnp.maximum(g1, g2)))

    def zero(i, _):
        s_v[pl.ds(i * 16, 16)] = jnp.zeros((16,), jnp.float32)
        return 0
    lax.fori_loop(0, (NH * NP) // 16, zero, 0)

    base_w = wid * EW
    for c in range(NCHUNK):
        base = base_w + c * CHUNK
        pltpu.sync_copy(src_hbm.at[pl.ds(base, CHUNK)],
                        src_v.at[pl.ds(0, CHUNK)])
        for h in range(NH):
            pltpu.sync_copy(ap_hbms[h].at[pl.ds(base, CHUNK)],
                            ap_vs[h].at[pl.ds(0, CHUNK)])

        def group(g, _):
            off = g * 16
            mask = iota < (CHUNK - off)
            src16 = src_v[pl.ds(off, 16)]
            src16 = jnp.minimum(jnp.maximum(src16, 0), N_NODES - 1)
            for h in range(NH):
                att = ap_vs[h][pl.ds(off, 16)]
                w = jnp.exp(att - m_sc[h])
                plsc.addupdate_scatter(
                    s_v, [h * NP + src16], w, mask=mask)
            return 0

        lax.fori_loop(0, NGROUP, group, 0)

    pltpu.sync_copy(s_v, s_hbm.at[pl.ds(wid * NH * NP, NH * NP)])


def _segment_sums(ap, src, m_part):
    mesh = plsc.VectorSubcoreMesh(core_axis_name="c", subcore_axis_name="s")
    kern = functools.partial(
        pl.kernel,
        out_type=jax.ShapeDtypeStruct((NW * NH * NP,), jnp.float32),
        mesh=mesh,
        scratch_types=[
            pltpu.VMEM((NH * NP,), jnp.float32),
            pltpu.VMEM((1008,), jnp.float32),
            pltpu.VMEM((1008,), jnp.float32),
            pltpu.VMEM((1008,), jnp.float32),
            pltpu.VMEM((1008,), jnp.float32),
            pltpu.VMEM((1008,), jnp.int32),
            pltpu.VMEM((NW * 16,), jnp.float32),
        ],
        compiler_params=pltpu.CompilerParams(needs_layout_passes=False),
    )(_segsum_body)
    return kern(ap[0], ap[1], ap[2], ap[3], src, m_part)


def _out_body(h_ref, s_ref, msel_ref, out_ref):
    acc = s_ref[0]
    for i in range(1, NW):
        acc = acc + s_ref[i]
    s_exp = lax.dot_general(
        acc, msel_ref[...],
        (((0,), (0,)), ((), ())),
        preferred_element_type=jnp.float32,
    )
    out_ref[...] = h_ref[...] * (s_exp / (s_exp + jnp.float32(1e-8)))


def _combine(H, s_parts, msel):
    blk = 2048
    grid = NP // blk
    return pl.pallas_call(
        _out_body,
        grid=(grid,),
        in_specs=[
            pl.BlockSpec((blk, D), lambda i: (i, 0)),
            pl.BlockSpec((NW, NH, blk), lambda i: (0, 0, i)),
            pl.BlockSpec((NH, D), lambda i: (0, 0)),
        ],
        out_specs=pl.BlockSpec((blk, D), lambda i: (i, 0)),
        out_shape=jax.ShapeDtypeStruct((NP, D), jnp.float32),
    )(H, s_parts, msel)


@jax.jit
def kernel(V, E, edges, W_edge, W_att, b_att):
    V2 = V[0]
    E2 = E[0]
    src = edges[0, :, 0]
    dst = edges[0, :, 1]

    wcat = jnp.concatenate([W_edge[h] for h in range(NH)], axis=1)
    wsr = jnp.zeros((D, 8), jnp.float32)
    for h in range(NH):
        wsr = wsr.at[h * OPH:(h + 1) * OPH, h].set(W_att[h, :OPH, 0])
        wsr = wsr.at[h * OPH:(h + 1) * OPH, NH + h].set(
            W_att[h, OPH:2 * OPH, 0])
    bvec = jnp.zeros((1, 8), jnp.float32)
    bvec = bvec.at[0, :NH].set(b_att[:, 0])
    waep = jnp.zeros((8, D), jnp.float32)
    waep = waep.at[:NH].set(W_att[:, 2 * OPH:, 0])
    msel = jnp.zeros((NH, D), jnp.float32)
    for h in range(NH):
        msel = msel.at[h, h * OPH:(h + 1) * OPH].set(1.0)

    V2p = jnp.pad(V2, ((0, NP - N_NODES), (0, 0)))
    H, a_tab = _node_tables(V2p, wcat, wsr, bvec)
    e_sc = _edge_scores(E2, waep)

    a_flat = a_tab.reshape(NP * 8)
    outs = _edge_logits(a_flat, e_sc, src, dst)
    ap, m_part = outs[:NH], outs[NH]
    s_parts = _segment_sums(ap, src, m_part)

    out = _combine(H, s_parts.reshape(NW, NH, NP), msel)
    return out[:N_NODES][None]


# speedup vs baseline: 2515.8691x; 1.2778x over previous
import functools

import jax
import jax.numpy as jnp
from jax import lax
from jax.experimental import pallas as pl
from jax.experimental.pallas import tpu as pltpu
from jax.experimental.pallas import tpu_sc as plsc

N_NODES = 10000
NP = 10240
N_EDGES = 160000
D = 128
NH = 4
OPH = 32

NC = 2
NS = 16
NW = NC * NS
EW = N_EDGES // NW
CHUNK = 1000
NCHUNK = EW // CHUNK
NPAIR = 31
TAIL_OFF = NPAIR * 32

_NEG = -1e30


def _node_body(v_ref, wcat_ref, wsr_ref, bvec_ref, h_ref, a_ref):
    h = jnp.dot(v_ref[...], wcat_ref[...], preferred_element_type=jnp.float32)
    h_ref[...] = h
    a_ref[...] = (
        jnp.dot(h, wsr_ref[...], preferred_element_type=jnp.float32)
        + bvec_ref[...]
    )


def _node_tables(V2, wcat, wsr, bvec):
    blk = 2048
    grid = pl.cdiv(N_NODES, blk)
    return pl.pallas_call(
        _node_body,
        grid=(grid,),
        in_specs=[
            pl.BlockSpec((blk, D), lambda i: (i, 0)),
            pl.BlockSpec((D, D), lambda i: (0, 0)),
            pl.BlockSpec((D, 8), lambda i: (0, 0)),
            pl.BlockSpec((1, 8), lambda i: (0, 0)),
        ],
        out_specs=[
            pl.BlockSpec((blk, D), lambda i: (i, 0)),
            pl.BlockSpec((blk, 8), lambda i: (i, 0)),
        ],
        out_shape=[
            jax.ShapeDtypeStruct((N_NODES, D), jnp.float32),
            jax.ShapeDtypeStruct((N_NODES, 8), jnp.float32),
        ],
    )(V2, wcat, wsr, bvec)


def _escore_body(waep_ref, e_ref, o0_ref, o1_ref, o2_ref, o3_ref):
    r = lax.dot_general(
        waep_ref[...], e_ref[...],
        (((1,), (1,)), ((), ())),
        preferred_element_type=jnp.float32,
    )
    for h, o_ref in enumerate((o0_ref, o1_ref, o2_ref, o3_ref)):
        o_ref[...] = r[h]


def _edge_scores(E2, waep):
    blk = 4096
    grid = pl.cdiv(N_EDGES, blk)
    return pl.pallas_call(
        _escore_body,
        grid=(grid,),
        in_specs=[
            pl.BlockSpec((8, D), lambda i: (0, 0)),
            pl.BlockSpec((blk, D), lambda i: (i, 0)),
        ],
        out_specs=[pl.BlockSpec((blk,), lambda i: (i,))] * NH,
        out_shape=[jax.ShapeDtypeStruct((N_EDGES,), jnp.float32)] * NH,
    )(waep, E2)


def _logits_body(a_hbm, e0_hbm, e1_hbm, e2_hbm, e3_hbm, src_hbm, dst_hbm,
                 ap0_hbm, ap1_hbm, ap2_hbm, ap3_hbm, m_hbm,
                 a_v,
                 src_v0, dst_v0, ea0_v0, ea1_v0, ea2_v0, ea3_v0,
                 src_v1, dst_v1, ea0_v1, ea1_v1, ea2_v1, ea3_v1,
                 stage_v, ld_sem0, ld_sem1, st_sem0, st_sem1):
    e_hbms = (e0_hbm, e1_hbm, e2_hbm, e3_hbm)
    ap_hbms = (ap0_hbm, ap1_hbm, ap2_hbm, ap3_hbm)
    slots = (
        (src_v0, dst_v0, (ea0_v0, ea1_v0, ea2_v0, ea3_v0), ld_sem0, st_sem0),
        (src_v1, dst_v1, (ea0_v1, ea1_v1, ea2_v1, ea3_v1), ld_sem1, st_sem1),
    )
    cid = lax.axis_index("c")
    sid = lax.axis_index("s")
    wid = sid * NC + cid
    iota = lax.iota(jnp.int32, 16)

    pltpu.sync_copy(a_hbm, a_v)
    base_w = wid * EW

    def start_loads(c):
        src_v, dst_v, e_vs, ld_sem, _ = slots[c % 2]
        base = base_w + c * CHUNK
        ds = []
        ds.append(pltpu.async_copy(src_hbm.at[pl.ds(base, CHUNK)],
                                   src_v.at[pl.ds(0, CHUNK)], ld_sem))
        ds.append(pltpu.async_copy(dst_hbm.at[pl.ds(base, CHUNK)],
                                   dst_v.at[pl.ds(0, CHUNK)], ld_sem))
        for h in range(NH):
            ds.append(pltpu.async_copy(e_hbms[h].at[pl.ds(base, CHUNK)],
                                       e_vs[h].at[pl.ds(0, CHUNK)], ld_sem))
        return ds

    def start_stores(c):
        _, _, e_vs, _, st_sem = slots[c % 2]
        base = base_w + c * CHUNK
        return [
            pltpu.async_copy(e_vs[h].at[pl.ds(0, CHUNK)],
                             ap_hbms[h].at[pl.ds(base, CHUNK)], st_sem)
            for h in range(NH)
        ]

    def one_group(src_v, dst_v, e_vs, off, carry, mask=None):
        src16 = src_v[pl.ds(off, 16)]
        dst16 = dst_v[pl.ds(off, 16)]
        src16 = jnp.minimum(jnp.maximum(src16, 0), N_NODES - 1)
        dst16 = jnp.minimum(jnp.maximum(dst16, 0), N_NODES - 1)
        out = []
        for h in range(NH):
            a_s = plsc.load_gather(a_v, [src16 * 8 + h])
            a_r = plsc.load_gather(a_v, [dst16 * 8 + (NH + h)])
            x = a_s + a_r + e_vs[h][pl.ds(off, 16)]
            att = jnp.where(x >= 0, x, x * jnp.float32(0.2))
            if mask is not None:
                att = jnp.where(mask, att, _NEG)
            e_vs[h][pl.ds(off, 16)] = att
            out.append(jnp.maximum(carry[h], att))
        return tuple(out)

    vmaxs = (jnp.full((16,), _NEG),) * NH
    loads = {0: start_loads(0)}
    stores = {}
    for c in range(NCHUNK):
        src_v, dst_v, e_vs, _, _ = slots[c % 2]
        if c + 1 < NCHUNK:
            if c - 1 >= 0:
                for d in stores[c - 1]:
                    d.wait()
            loads[c + 1] = start_loads(c + 1)
        for d in loads[c]:
            d.wait()

        def pair(i, carry):
            carry = one_group(src_v, dst_v, e_vs, i * 32, carry)
            return one_group(src_v, dst_v, e_vs, i * 32 + 16, carry)

        vmaxs = lax.fori_loop(0, NPAIR, pair, vmaxs)
        vmaxs = one_group(src_v, dst_v, e_vs, TAIL_OFF, vmaxs,
                          mask=iota < (CHUNK - TAIL_OFF))
        stores[c] = start_stores(c)

    for c in (NCHUNK - 2, NCHUNK - 1):
        for d in stores[c]:
            d.wait()

    comb = jnp.full((16,), _NEG)
    for h in range(NH):
        comb = jnp.where(iota == h, jnp.max(vmaxs[h]), comb)
    stage_v[...] = comb
    pltpu.sync_copy(stage_v, m_hbm.at[pl.ds(wid * 16, 16)])


def _edge_logits(a_tab, e_sc, src, dst):
    mesh = plsc.VectorSubcoreMesh(core_axis_name="c", subcore_axis_name="s")
    chunk_bufs = [
        pltpu.VMEM((1008,), jnp.int32),
        pltpu.VMEM((1008,), jnp.int32),
        pltpu.VMEM((1008,), jnp.float32),
        pltpu.VMEM((1008,), jnp.float32),
        pltpu.VMEM((1008,), jnp.float32),
        pltpu.VMEM((1008,), jnp.float32),
    ]
    kern = functools.partial(
        pl.kernel,
        out_type=[jax.ShapeDtypeStruct((N_EDGES,), jnp.float32)] * NH
        + [jax.ShapeDtypeStruct((NW * 16,), jnp.float32)],
        mesh=mesh,
        scratch_types=[pltpu.VMEM((N_NODES * 8,), jnp.float32)]
        + chunk_bufs + chunk_bufs
        + [pltpu.VMEM((16,), jnp.float32)]
        + [pltpu.SemaphoreType.DMA] * 4,
        compiler_params=pltpu.CompilerParams(needs_layout_passes=False),
    )(_logits_body)
    return kern(a_tab, e_sc[0], e_sc[1], e_sc[2], e_sc[3], src, dst)


def _segsum_body(ap0_hbm, ap1_hbm, ap2_hbm, ap3_hbm, src_hbm, m_hbm, z_hbm,
                 s_hbm,
                 s_v,
                 src_v0, ap0_v0, ap1_v0, ap2_v0, ap3_v0,
                 src_v1, ap0_v1, ap1_v1, ap2_v1, ap3_v1,
                 m_v, ld_sem0, ld_sem1, z_sem):
    ap_hbms = (ap0_hbm, ap1_hbm, ap2_hbm, ap3_hbm)
    slots = (
        (src_v0, (ap0_v0, ap1_v0, ap2_v0, ap3_v0), ld_sem0),
        (src_v1, (ap0_v1, ap1_v1, ap2_v1, ap3_v1), ld_sem1),
    )
    cid = lax.axis_index("c")
    sid = lax.axis_index("s")
    wid = sid * NC + cid
    iota = lax.iota(jnp.int32, 16)

    zd = pltpu.async_copy(z_hbm, s_v, z_sem)
    pltpu.sync_copy(m_hbm, m_v)
    m_sc = []
    for h in range(NH):
        g1 = plsc.load_gather(m_v, [iota * 16 + h])
        g2 = plsc.load_gather(m_v, [(iota + 16) * 16 + h])
        m_sc.append(jnp.max(jnp.maximum(g1, g2)))

    base_w = wid * EW

    def start_loads(c):
        src_v, ap_vs, ld_sem = slots[c % 2]
        base = base_w + c * CHUNK
        ds = [pltpu.async_copy(src_hbm.at[pl.ds(base, CHUNK)],
                               src_v.at[pl.ds(0, CHUNK)], ld_sem)]
        for h in range(NH):
            ds.append(pltpu.async_copy(ap_hbms[h].at[pl.ds(base, CHUNK)],
                                       ap_vs[h].at[pl.ds(0, CHUNK)], ld_sem))
        return ds

    def one_group(src_v, ap_vs, off, mask=None):
        src16 = src_v[pl.ds(off, 16)]
        src16 = jnp.minimum(jnp.maximum(src16, 0), N_NODES - 1)
        for h in range(NH):
            att = ap_vs[h][pl.ds(off, 16)]
            w = jnp.exp(att - m_sc[h])
            plsc.addupdate_scatter(s_v, [h * NP + src16], w, mask=mask)

    loads = {0: start_loads(0)}
    zd.wait()
    for c in range(NCHUNK):
        src_v, ap_vs, _ = slots[c % 2]
        if c + 1 < NCHUNK:
            loads[c + 1] = start_loads(c + 1)
        for d in loads[c]:
            d.wait()

        def pair(i, _):
            one_group(src_v, ap_vs, i * 32)
            one_group(src_v, ap_vs, i * 32 + 16)
            return 0

        lax.fori_loop(0, NPAIR, pair, 0)
        one_group(src_v, ap_vs, TAIL_OFF, mask=iota < (CHUNK - TAIL_OFF))

    pltpu.sync_copy(s_v, s_hbm.at[pl.ds(wid * NH * NP, NH * NP)])


def _segment_sums(ap, src, m_part, zeros):
    mesh = plsc.VectorSubcoreMesh(core_axis_name="c", subcore_axis_name="s")
    chunk_bufs = [
        pltpu.VMEM((1008,), jnp.int32),
        pltpu.VMEM((1008,), jnp.float32),
        pltpu.VMEM((1008,), jnp.float32),
        pltpu.VMEM((1008,), jnp.float32),
        pltpu.VMEM((1008,), jnp.float32),
    ]
    kern = functools.partial(
        pl.kernel,
        out_type=jax.ShapeDtypeStruct((NW * NH * NP,), jnp.float32),
        mesh=mesh,
        scratch_types=[pltpu.VMEM((NH * NP,), jnp.float32)]
        + chunk_bufs + chunk_bufs
        + [pltpu.VMEM((NW * 16,), jnp.float32)]
        + [pltpu.SemaphoreType.DMA] * 3,
        compiler_params=pltpu.CompilerParams(needs_layout_passes=False),
    )(_segsum_body)
    return kern(ap[0], ap[1], ap[2], ap[3], src, m_part, zeros)


def _out_body(h_ref, s_ref, msel_ref, out_ref):
    acc = s_ref[0]
    for i in range(1, NW):
        acc = acc + s_ref[i]
    s_exp = lax.dot_general(
        acc, msel_ref[...],
        (((0,), (0,)), ((), ())),
        preferred_element_type=jnp.float32,
    )
    out_ref[...] = h_ref[...] * (s_exp / (s_exp + jnp.float32(1e-8)))


def _combine(H, s_parts, msel):
    blk = 2048
    grid = pl.cdiv(N_NODES, blk)
    return pl.pallas_call(
        _out_body,
        grid=(grid,),
        in_specs=[
            pl.BlockSpec((blk, D), lambda i: (i, 0)),
            pl.BlockSpec((NW, NH, blk), lambda i: (0, 0, i)),
            pl.BlockSpec((NH, D), lambda i: (0, 0)),
        ],
        out_specs=pl.BlockSpec((blk, D), lambda i: (i, 0)),
        out_shape=jax.ShapeDtypeStruct((N_NODES, D), jnp.float32),
    )(H, s_parts, msel)


@jax.jit
def kernel(V, E, edges, W_edge, W_att, b_att):
    V2 = V[0]
    E2 = E[0]
    src = edges[0, :, 0]
    dst = edges[0, :, 1]

    wcat = jnp.concatenate([W_edge[h] for h in range(NH)], axis=1)
    wsr = jnp.zeros((D, 8), jnp.float32)
    for h in range(NH):
        wsr = wsr.at[h * OPH:(h + 1) * OPH, h].set(W_att[h, :OPH, 0])
        wsr = wsr.at[h * OPH:(h + 1) * OPH, NH + h].set(
            W_att[h, OPH:2 * OPH, 0])
    bvec = jnp.zeros((1, 8), jnp.float32)
    bvec = bvec.at[0, :NH].set(b_att[:, 0])
    waep = jnp.zeros((8, D), jnp.float32)
    waep = waep.at[:NH].set(W_att[:, 2 * OPH:, 0])
    msel = jnp.zeros((NH, D), jnp.float32)
    for h in range(NH):
        msel = msel.at[h, h * OPH:(h + 1) * OPH].set(1.0)
    zeros = jnp.zeros((NH * NP,), jnp.float32)

    H, a_tab = _node_tables(V2, wcat, wsr, bvec)
    e_sc = _edge_scores(E2, waep)

    a_flat = a_tab.reshape(N_NODES * 8)
    outs = _edge_logits(a_flat, e_sc, src, dst)
    ap, m_part = outs[:NH], outs[NH]
    s_parts = _segment_sums(ap, src, m_part, zeros)

    out = _combine(H, s_parts.reshape(NW, NH, NP), msel)
    return out[None]


# speedup vs baseline: 2525.6776x; 1.0039x over previous
import functools

import jax
import jax.numpy as jnp
from jax import lax
from jax.experimental import pallas as pl
from jax.experimental.pallas import tpu as pltpu
from jax.experimental.pallas import tpu_sc as plsc

N_NODES = 10000
NP = 10240
N_EDGES = 160000
D = 128
NH = 4
OPH = 32

NC = 2
NS = 16
NW = NC * NS
EW = N_EDGES // NW
CHUNK = 1000
NCHUNK = EW // CHUNK
NPAIR = 31
TAIL_OFF = NPAIR * 32

_NEG = -1e30


def _node_body(v_ref, wcat_ref, wsr_ref, bvec_ref, h_ref, a_ref):
    h = jnp.dot(v_ref[...], wcat_ref[...], preferred_element_type=jnp.float32)
    h_ref[...] = h
    a_ref[...] = (
        jnp.dot(h, wsr_ref[...], preferred_element_type=jnp.float32)
        + bvec_ref[...]
    )


def _node_tables(V2, wcat, wsr, bvec):
    blk = 2048
    grid = pl.cdiv(N_NODES, blk)
    return pl.pallas_call(
        _node_body,
        grid=(grid,),
        in_specs=[
            pl.BlockSpec((blk, D), lambda i: (i, 0)),
            pl.BlockSpec((D, D), lambda i: (0, 0)),
            pl.BlockSpec((D, 8), lambda i: (0, 0)),
            pl.BlockSpec((1, 8), lambda i: (0, 0)),
        ],
        out_specs=[
            pl.BlockSpec((blk, D), lambda i: (i, 0)),
            pl.BlockSpec((blk, 8), lambda i: (i, 0)),
        ],
        out_shape=[
            jax.ShapeDtypeStruct((N_NODES, D), jnp.float32),
            jax.ShapeDtypeStruct((N_NODES, 8), jnp.float32),
        ],
    )(V2, wcat, wsr, bvec)


def _escore_body(waep_ref, e_ref, o0_ref, o1_ref, o2_ref, o3_ref):
    r = lax.dot_general(
        waep_ref[...], e_ref[...],
        (((1,), (1,)), ((), ())),
        preferred_element_type=jnp.float32,
    )
    for h, o_ref in enumerate((o0_ref, o1_ref, o2_ref, o3_ref)):
        o_ref[...] = r[h]


def _edge_scores(E2, waep):
    blk = 4096
    grid = pl.cdiv(N_EDGES, blk)
    return pl.pallas_call(
        _escore_body,
        grid=(grid,),
        in_specs=[
            pl.BlockSpec((8, D), lambda i: (0, 0)),
            pl.BlockSpec((blk, D), lambda i: (i, 0)),
        ],
        out_specs=[pl.BlockSpec((blk,), lambda i: (i,))] * NH,
        out_shape=[jax.ShapeDtypeStruct((N_EDGES,), jnp.float32)] * NH,
    )(waep, E2)


def _logits_body(a_hbm, e0_hbm, e1_hbm, e2_hbm, e3_hbm, src_hbm, dst_hbm,
                 ap0_hbm, ap1_hbm, ap2_hbm, ap3_hbm, m_hbm,
                 a_v,
                 src_v0, dst_v0, ea0_v0, ea1_v0, ea2_v0, ea3_v0,
                 src_v1, dst_v1, ea0_v1, ea1_v1, ea2_v1, ea3_v1,
                 stage_v, ld_sem0, ld_sem1, st_sem0, st_sem1):
    e_hbms = (e0_hbm, e1_hbm, e2_hbm, e3_hbm)
    ap_hbms = (ap0_hbm, ap1_hbm, ap2_hbm, ap3_hbm)
    slots = (
        (src_v0, dst_v0, (ea0_v0, ea1_v0, ea2_v0, ea3_v0), ld_sem0, st_sem0),
        (src_v1, dst_v1, (ea0_v1, ea1_v1, ea2_v1, ea3_v1), ld_sem1, st_sem1),
    )
    cid = lax.axis_index("c")
    sid = lax.axis_index("s")
    wid = sid * NC + cid
    iota = lax.iota(jnp.int32, 16)

    pltpu.sync_copy(a_hbm, a_v)
    base_w = wid * EW

    def start_loads(c):
        src_v, dst_v, e_vs, ld_sem, _ = slots[c % 2]
        base = base_w + c * CHUNK
        ds = []
        ds.append(pltpu.async_copy(src_hbm.at[pl.ds(base, CHUNK)],
                                   src_v.at[pl.ds(0, CHUNK)], ld_sem))
        ds.append(pltpu.async_copy(dst_hbm.at[pl.ds(base, CHUNK)],
                                   dst_v.at[pl.ds(0, CHUNK)], ld_sem))
        for h in range(NH):
            ds.append(pltpu.async_copy(e_hbms[h].at[pl.ds(base, CHUNK)],
                                       e_vs[h].at[pl.ds(0, CHUNK)], ld_sem))
        return ds

    def start_stores(c):
        _, _, e_vs, _, st_sem = slots[c % 2]
        base = base_w + c * CHUNK
        return [
            pltpu.async_copy(e_vs[h].at[pl.ds(0, CHUNK)],
                             ap_hbms[h].at[pl.ds(base, CHUNK)], st_sem)
            for h in range(NH)
        ]

    def one_group(src_v, dst_v, e_vs, off, carry, mask=None):
        src16 = src_v[pl.ds(off, 16)]
        dst16 = dst_v[pl.ds(off, 16)]
        src16 = jnp.minimum(jnp.maximum(src16, 0), N_NODES - 1)
        dst16 = jnp.minimum(jnp.maximum(dst16, 0), N_NODES - 1)
        out = []
        for h in range(NH):
            a_s = plsc.load_gather(a_v, [src16 * 8 + h])
            a_r = plsc.load_gather(a_v, [dst16 * 8 + (NH + h)])
            x = a_s + a_r + e_vs[h][pl.ds(off, 16)]
            att = jnp.maximum(x, x * jnp.float32(0.2))
            if mask is not None:
                att = jnp.where(mask, att, _NEG)
            e_vs[h][pl.ds(off, 16)] = att
            out.append(jnp.maximum(carry[h], att))
        return tuple(out)

    vmaxs = (jnp.full((16,), _NEG),) * NH
    loads = {0: start_loads(0)}
    stores = {}
    for c in range(NCHUNK):
        src_v, dst_v, e_vs, _, _ = slots[c % 2]
        if c + 1 < NCHUNK:
            if c - 1 >= 0:
                for d in stores[c - 1]:
                    d.wait()
            loads[c + 1] = start_loads(c + 1)
        for d in loads[c]:
            d.wait()

        def pair(i, carry):
            carry = one_group(src_v, dst_v, e_vs, i * 32, carry)
            return one_group(src_v, dst_v, e_vs, i * 32 + 16, carry)

        vmaxs = lax.fori_loop(0, NPAIR, pair, vmaxs)
        vmaxs = one_group(src_v, dst_v, e_vs, TAIL_OFF, vmaxs,
                          mask=iota < (CHUNK - TAIL_OFF))
        stores[c] = start_stores(c)

    for c in (NCHUNK - 2, NCHUNK - 1):
        for d in stores[c]:
            d.wait()

    comb = jnp.full((16,), _NEG)
    for h in range(NH):
        comb = jnp.where(iota == h, jnp.max(vmaxs[h]), comb)
    stage_v[...] = comb
    pltpu.sync_copy(stage_v, m_hbm.at[pl.ds(wid * 16, 16)])


def _edge_logits(a_tab, e_sc, src, dst):
    mesh = plsc.VectorSubcoreMesh(core_axis_name="c", subcore_axis_name="s")
    chunk_bufs = [
        pltpu.VMEM((1008,), jnp.int32),
        pltpu.VMEM((1008,), jnp.int32),
        pltpu.VMEM((1008,), jnp.float32),
        pltpu.VMEM((1008,), jnp.float32),
        pltpu.VMEM((1008,), jnp.float32),
        pltpu.VMEM((1008,), jnp.float32),
    ]
    kern = functools.partial(
        pl.kernel,
        out_type=[jax.ShapeDtypeStruct((N_EDGES,), jnp.float32)] * NH
        + [jax.ShapeDtypeStruct((NW * 16,), jnp.float32)],
        mesh=mesh,
        scratch_types=[pltpu.VMEM((N_NODES * 8,), jnp.float32)]
        + chunk_bufs + chunk_bufs
        + [pltpu.VMEM((16,), jnp.float32)]
        + [pltpu.SemaphoreType.DMA] * 4,
        compiler_params=pltpu.CompilerParams(needs_layout_passes=False),
    )(_logits_body)
    return kern(a_tab, e_sc[0], e_sc[1], e_sc[2], e_sc[3], src, dst)


def _segsum_body(ap0_hbm, ap1_hbm, ap2_hbm, ap3_hbm, src_hbm, m_hbm, z_hbm,
                 s_hbm,
                 s_v,
                 src_v0, ap0_v0, ap1_v0, ap2_v0, ap3_v0,
                 src_v1, ap0_v1, ap1_v1, ap2_v1, ap3_v1,
                 m_v, ld_sem0, ld_sem1, z_sem):
    ap_hbms = (ap0_hbm, ap1_hbm, ap2_hbm, ap3_hbm)
    slots = (
        (src_v0, (ap0_v0, ap1_v0, ap2_v0, ap3_v0), ld_sem0),
        (src_v1, (ap0_v1, ap1_v1, ap2_v1, ap3_v1), ld_sem1),
    )
    cid = lax.axis_index("c")
    sid = lax.axis_index("s")
    wid = sid * NC + cid
    iota = lax.iota(jnp.int32, 16)

    zd = pltpu.async_copy(z_hbm, s_v, z_sem)
    pltpu.sync_copy(m_hbm, m_v)
    m_sc = []
    for h in range(NH):
        g1 = plsc.load_gather(m_v, [iota * 16 + h])
        g2 = plsc.load_gather(m_v, [(iota + 16) * 16 + h])
        m_sc.append(jnp.max(jnp.maximum(g1, g2)))

    base_w = wid * EW

    def start_loads(c):
        src_v, ap_vs, ld_sem = slots[c % 2]
        base = base_w + c * CHUNK
        ds = [pltpu.async_copy(src_hbm.at[pl.ds(base, CHUNK)],
                               src_v.at[pl.ds(0, CHUNK)], ld_sem)]
        for h in range(NH):
            ds.append(pltpu.async_copy(ap_hbms[h].at[pl.ds(base, CHUNK)],
                                       ap_vs[h].at[pl.ds(0, CHUNK)], ld_sem))
        return ds

    def one_group(src_v, ap_vs, off, mask=None):
        src16 = src_v[pl.ds(off, 16)]
        src16 = jnp.minimum(jnp.maximum(src16, 0), N_NODES - 1)
        for h in range(NH):
            att = ap_vs[h][pl.ds(off, 16)]
            w = jnp.exp(att - m_sc[h])
            plsc.addupdate_scatter(s_v, [h * NP + src16], w, mask=mask)

    loads = {0: start_loads(0)}
    zd.wait()
    for c in range(NCHUNK):
        src_v, ap_vs, _ = slots[c % 2]
        if c + 1 < NCHUNK:
            loads[c + 1] = start_loads(c + 1)
        for d in loads[c]:
            d.wait()

        def pair(i, _):
            one_group(src_v, ap_vs, i * 32)
            one_group(src_v, ap_vs, i * 32 + 16)
            return 0

        lax.fori_loop(0, NPAIR, pair, 0)
        one_group(src_v, ap_vs, TAIL_OFF, mask=iota < (CHUNK - TAIL_OFF))

    pltpu.sync_copy(s_v, s_hbm.at[pl.ds(wid * NH * NP, NH * NP)])


def _segment_sums(ap, src, m_part, zeros):
    mesh = plsc.VectorSubcoreMesh(core_axis_name="c", subcore_axis_name="s")
    chunk_bufs = [
        pltpu.VMEM((1008,), jnp.int32),
        pltpu.VMEM((1008,), jnp.float32),
        pltpu.VMEM((1008,), jnp.float32),
        pltpu.VMEM((1008,), jnp.float32),
        pltpu.VMEM((1008,), jnp.float32),
    ]
    kern = functools.partial(
        pl.kernel,
        out_type=jax.ShapeDtypeStruct((NW * NH * NP,), jnp.float32),
        mesh=mesh,
        scratch_types=[pltpu.VMEM((NH * NP,), jnp.float32)]
        + chunk_bufs + chunk_bufs
        + [pltpu.VMEM((NW * 16,), jnp.float32)]
        + [pltpu.SemaphoreType.DMA] * 3,
        compiler_params=pltpu.CompilerParams(needs_layout_passes=False),
    )(_segsum_body)
    return kern(ap[0], ap[1], ap[2], ap[3], src, m_part, zeros)


def _out_body(h_ref, s_ref, rsel_ref, msel_ref, out_ref):
    acc = jnp.dot(rsel_ref[...], s_ref[...],
                  preferred_element_type=jnp.float32)
    s_exp = lax.dot_general(
        acc, msel_ref[...],
        (((0,), (0,)), ((), ())),
        preferred_element_type=jnp.float32,
    )
    out_ref[...] = h_ref[...] * (s_exp / (s_exp + jnp.float32(1e-8)))


def _combine(H, s_parts, rsel, msel):
    blk = 2048
    grid = pl.cdiv(N_NODES, blk)
    return pl.pallas_call(
        _out_body,
        grid=(grid,),
        in_specs=[
            pl.BlockSpec((blk, D), lambda i: (i, 0)),
            pl.BlockSpec((NW * NH, blk), lambda i: (0, i)),
            pl.BlockSpec((NH, NW * NH), lambda i: (0, 0)),
            pl.BlockSpec((NH, D), lambda i: (0, 0)),
        ],
        out_specs=pl.BlockSpec((blk, D), lambda i: (i, 0)),
        out_shape=jax.ShapeDtypeStruct((N_NODES, D), jnp.float32),
    )(H, s_parts, rsel, msel)


@jax.jit
def kernel(V, E, edges, W_edge, W_att, b_att):
    V2 = V[0]
    E2 = E[0]
    src = edges[0, :, 0]
    dst = edges[0, :, 1]

    wcat = jnp.concatenate([W_edge[h] for h in range(NH)], axis=1)
    wsr = jnp.zeros((D, 8), jnp.float32)
    for h in range(NH):
        wsr = wsr.at[h * OPH:(h + 1) * OPH, h].set(W_att[h, :OPH, 0])
        wsr = wsr.at[h * OPH:(h + 1) * OPH, NH + h].set(
            W_att[h, OPH:2 * OPH, 0])
    bvec = jnp.zeros((1, 8), jnp.float32)
    bvec = bvec.at[0, :NH].set(b_att[:, 0])
    waep = jnp.zeros((8, D), jnp.float32)
    waep = waep.at[:NH].set(W_att[:, 2 * OPH:, 0])
    msel = jnp.zeros((NH, D), jnp.float32)
    for h in range(NH):
        msel = msel.at[h, h * OPH:(h + 1) * OPH].set(1.0)
    rsel = jnp.tile(jnp.eye(NH, dtype=jnp.float32), (1, NW))
    zeros = jnp.zeros((NH * NP,), jnp.float32)

    H, a_tab = _node_tables(V2, wcat, wsr, bvec)
    e_sc = _edge_scores(E2, waep)

    a_flat = a_tab.reshape(N_NODES * 8)
    outs = _edge_logits(a_flat, e_sc, src, dst)
    ap, m_part = outs[:NH], outs[NH]
    s_parts = _segment_sums(ap, src, m_part, zeros)

    out = _combine(H, s_parts.reshape(NW * NH, NP), rsel, msel)
    return out[None]


# speedup vs baseline: 2640.3313x; 1.0454x over previous
import functools

import jax
import jax.numpy as jnp
import numpy as np
from jax import lax
from jax.experimental import pallas as pl
from jax.experimental.pallas import tpu as pltpu
from jax.experimental.pallas import tpu_sc as plsc

N_NODES = 10000
NP = 10240
N_EDGES = 160000
D = 128
NH = 4
OPH = 32

NC = 2
NS = 16
NW = NC * NS
EW = N_EDGES // NW
CHUNK = 1000
NCHUNK = EW // CHUNK
NPAIR = 31
TAIL_OFF = NPAIR * 32

_NEG = -1e30


def _node_body(v_ref, wcat_ref, wsr_ref, bvec_ref, h_ref, a_ref):
    h = jnp.dot(v_ref[...], wcat_ref[...], preferred_element_type=jnp.float32)
    h_ref[...] = h
    a_ref[...] = (
        jnp.dot(h, wsr_ref[...], preferred_element_type=jnp.float32)
        + bvec_ref[...]
    )


def _node_tables(V2, wcat, wsr, bvec):
    blk = 2048
    grid = pl.cdiv(N_NODES, blk)
    return pl.pallas_call(
        _node_body,
        grid=(grid,),
        in_specs=[
            pl.BlockSpec((blk, D), lambda i: (i, 0)),
            pl.BlockSpec((D, D), lambda i: (0, 0)),
            pl.BlockSpec((D, 8), lambda i: (0, 0)),
            pl.BlockSpec((1, 8), lambda i: (0, 0)),
        ],
        out_specs=[
            pl.BlockSpec((blk, D), lambda i: (i, 0)),
            pl.BlockSpec((blk, 8), lambda i: (i, 0)),
        ],
        out_shape=[
            jax.ShapeDtypeStruct((N_NODES, D), jnp.float32),
            jax.ShapeDtypeStruct((N_NODES, 8), jnp.float32),
        ],
    )(V2, wcat, wsr, bvec)


def _escore_body(waep_ref, e_ref, o0_ref, o1_ref, o2_ref, o3_ref):
    r = lax.dot_general(
        waep_ref[...], e_ref[...],
        (((1,), (1,)), ((), ())),
        preferred_element_type=jnp.float32,
    )
    for h, o_ref in enumerate((o0_ref, o1_ref, o2_ref, o3_ref)):
        o_ref[...] = r[h]


def _edge_scores(E2, waep):
    blk = 4096
    grid = pl.cdiv(N_EDGES, blk)
    return pl.pallas_call(
        _escore_body,
        grid=(grid,),
        in_specs=[
            pl.BlockSpec((8, D), lambda i: (0, 0)),
            pl.BlockSpec((blk, D), lambda i: (i, 0)),
        ],
        out_specs=[pl.BlockSpec((blk,), lambda i: (i,))] * NH,
        out_shape=[jax.ShapeDtypeStruct((N_EDGES,), jnp.float32)] * NH,
    )(waep, E2)


def _logits_body(a_hbm, e0_hbm, e1_hbm, e2_hbm, e3_hbm, src_hbm, dst_hbm,
                 ap0_hbm, ap1_hbm, ap2_hbm, ap3_hbm, m_hbm,
                 a_v,
                 src_v0, dst_v0, ea0_v0, ea1_v0, ea2_v0, ea3_v0,
                 src_v1, dst_v1, ea0_v1, ea1_v1, ea2_v1, ea3_v1,
                 stage_v, ld_sem0, ld_sem1, st_sem0, st_sem1):
    e_hbms = (e0_hbm, e1_hbm, e2_hbm, e3_hbm)
    ap_hbms = (ap0_hbm, ap1_hbm, ap2_hbm, ap3_hbm)
    slots = (
        (src_v0, dst_v0, (ea0_v0, ea1_v0, ea2_v0, ea3_v0), ld_sem0, st_sem0),
        (src_v1, dst_v1, (ea0_v1, ea1_v1, ea2_v1, ea3_v1), ld_sem1, st_sem1),
    )
    cid = lax.axis_index("c")
    sid = lax.axis_index("s")
    wid = sid * NC + cid
    iota = lax.iota(jnp.int32, 16)

    pltpu.sync_copy(a_hbm, a_v)
    base_w = wid * EW

    def start_loads(c):
        src_v, dst_v, e_vs, ld_sem, _ = slots[c % 2]
        base = base_w + c * CHUNK
        ds = []
        ds.append(pltpu.async_copy(src_hbm.at[pl.ds(base, CHUNK)],
                                   src_v.at[pl.ds(0, CHUNK)], ld_sem))
        ds.append(pltpu.async_copy(dst_hbm.at[pl.ds(base, CHUNK)],
                                   dst_v.at[pl.ds(0, CHUNK)], ld_sem))
        for h in range(NH):
            ds.append(pltpu.async_copy(e_hbms[h].at[pl.ds(base, CHUNK)],
                                       e_vs[h].at[pl.ds(0, CHUNK)], ld_sem))
        return ds

    def start_stores(c):
        _, _, e_vs, _, st_sem = slots[c % 2]
        base = base_w + c * CHUNK
        return [
            pltpu.async_copy(e_vs[h].at[pl.ds(0, CHUNK)],
                             ap_hbms[h].at[pl.ds(base, CHUNK)], st_sem)
            for h in range(NH)
        ]

    def one_group(src_v, dst_v, e_vs, off, carry, mask=None):
        src16 = src_v[pl.ds(off, 16)]
        dst16 = dst_v[pl.ds(off, 16)]
        if mask is not None:
            src16 = jnp.minimum(jnp.maximum(src16, 0), N_NODES - 1)
            dst16 = jnp.minimum(jnp.maximum(dst16, 0), N_NODES - 1)
        out = []
        for h in range(NH):
            a_s = plsc.load_gather(a_v, [src16 * 8 + h])
            a_r = plsc.load_gather(a_v, [dst16 * 8 + (NH + h)])
            x = a_s + a_r + e_vs[h][pl.ds(off, 16)]
            att = jnp.maximum(x, x * jnp.float32(0.2))
            if mask is not None:
                att = jnp.where(mask, att, _NEG)
            e_vs[h][pl.ds(off, 16)] = att
            out.append(jnp.maximum(carry[h], att))
        return tuple(out)

    vmaxs = (jnp.full((16,), _NEG),) * NH
    loads = {0: start_loads(0)}
    stores = {}
    for c in range(NCHUNK):
        src_v, dst_v, e_vs, _, _ = slots[c % 2]
        if c + 1 < NCHUNK:
            if c - 1 >= 0:
                for d in stores[c - 1]:
                    d.wait()
            loads[c + 1] = start_loads(c + 1)
        for d in loads[c]:
            d.wait()

        def pair(i, carry):
            carry = one_group(src_v, dst_v, e_vs, i * 32, carry)
            return one_group(src_v, dst_v, e_vs, i * 32 + 16, carry)

        vmaxs = lax.fori_loop(0, NPAIR, pair, vmaxs)
        vmaxs = one_group(src_v, dst_v, e_vs, TAIL_OFF, vmaxs,
                          mask=iota < (CHUNK - TAIL_OFF))
        stores[c] = start_stores(c)

    for c in (NCHUNK - 2, NCHUNK - 1):
        for d in stores[c]:
            d.wait()

    comb = jnp.full((16,), _NEG)
    for h in range(NH):
        comb = jnp.where(iota == h, jnp.max(vmaxs[h]), comb)
    stage_v[...] = comb
    pltpu.sync_copy(stage_v, m_hbm.at[pl.ds(wid * 16, 16)])


def _edge_logits(a_tab, e_sc, src, dst):
    mesh = plsc.VectorSubcoreMesh(core_axis_name="c", subcore_axis_name="s")
    chunk_bufs = [
        pltpu.VMEM((1008,), jnp.int32),
        pltpu.VMEM((1008,), jnp.int32),
        pltpu.VMEM((1008,), jnp.float32),
        pltpu.VMEM((1008,), jnp.float32),
        pltpu.VMEM((1008,), jnp.float32),
        pltpu.VMEM((1008,), jnp.float32),
    ]
    kern = functools.partial(
        pl.kernel,
        out_type=[jax.ShapeDtypeStruct((N_EDGES,), jnp.float32)] * NH
        + [jax.ShapeDtypeStruct((NW * 16,), jnp.float32)],
        mesh=mesh,
        scratch_types=[pltpu.VMEM((N_NODES * 8,), jnp.float32)]
        + chunk_bufs + chunk_bufs
        + [pltpu.VMEM((16,), jnp.float32)]
        + [pltpu.SemaphoreType.DMA] * 4,
        compiler_params=pltpu.CompilerParams(needs_layout_passes=False),
    )(_logits_body)
    return kern(a_tab, e_sc[0], e_sc[1], e_sc[2], e_sc[3], src, dst)


def _segsum_body(ap0_hbm, ap1_hbm, ap2_hbm, ap3_hbm, src_hbm, m_hbm, z_hbm,
                 s_hbm,
                 s_v,
                 src_v0, ap0_v0, ap1_v0, ap2_v0, ap3_v0,
                 src_v1, ap0_v1, ap1_v1, ap2_v1, ap3_v1,
                 m_v, ld_sem0, ld_sem1, z_sem):
    ap_hbms = (ap0_hbm, ap1_hbm, ap2_hbm, ap3_hbm)
    slots = (
        (src_v0, (ap0_v0, ap1_v0, ap2_v0, ap3_v0), ld_sem0),
        (src_v1, (ap0_v1, ap1_v1, ap2_v1, ap3_v1), ld_sem1),
    )
    cid = lax.axis_index("c")
    sid = lax.axis_index("s")
    wid = sid * NC + cid
    iota = lax.iota(jnp.int32, 16)

    zd = pltpu.async_copy(z_hbm, s_v, z_sem)
    pltpu.sync_copy(m_hbm, m_v)
    m_sc = []
    for h in range(NH):
        g1 = plsc.load_gather(m_v, [iota * 16 + h])
        g2 = plsc.load_gather(m_v, [(iota + 16) * 16 + h])
        m_sc.append(jnp.max(jnp.maximum(g1, g2)))

    base_w = wid * EW

    def start_loads(c):
        src_v, ap_vs, ld_sem = slots[c % 2]
        base = base_w + c * CHUNK
        ds = [pltpu.async_copy(src_hbm.at[pl.ds(base, CHUNK)],
                               src_v.at[pl.ds(0, CHUNK)], ld_sem)]
        for h in range(NH):
            ds.append(pltpu.async_copy(ap_hbms[h].at[pl.ds(base, CHUNK)],
                                       ap_vs[h].at[pl.ds(0, CHUNK)], ld_sem))
        return ds

    def one_group(src_v, ap_vs, off, mask=None):
        src16 = src_v[pl.ds(off, 16)]
        if mask is not None:
            src16 = jnp.minimum(jnp.maximum(src16, 0), N_NODES - 1)
        for h in range(NH):
            att = ap_vs[h][pl.ds(off, 16)]
            w = jnp.exp(att - m_sc[h])
            plsc.addupdate_scatter(s_v, [h * NP + src16], w, mask=mask)

    loads = {0: start_loads(0)}
    zd.wait()
    for c in range(NCHUNK):
        src_v, ap_vs, _ = slots[c % 2]
        if c + 1 < NCHUNK:
            loads[c + 1] = start_loads(c + 1)
        for d in loads[c]:
            d.wait()

        def pair(i, _):
            one_group(src_v, ap_vs, i * 32)
            one_group(src_v, ap_vs, i * 32 + 16)
            return 0

        lax.fori_loop(0, NPAIR, pair, 0)
        one_group(src_v, ap_vs, TAIL_OFF, mask=iota < (CHUNK - TAIL_OFF))

    pltpu.sync_copy(s_v, s_hbm.at[pl.ds(wid * NH * NP, NH * NP)])


def _segment_sums(ap, src, m_part, zeros):
    mesh = plsc.VectorSubcoreMesh(core_axis_name="c", subcore_axis_name="s")
    chunk_bufs = [
        pltpu.VMEM((1008,), jnp.int32),
        pltpu.VMEM((1008,), jnp.float32),
        pltpu.VMEM((1008,), jnp.float32),
        pltpu.VMEM((1008,), jnp.float32),
        pltpu.VMEM((1008,), jnp.float32),
    ]
    kern = functools.partial(
        pl.kernel,
        out_type=jax.ShapeDtypeStruct((NW * NH * NP,), jnp.float32),
        mesh=mesh,
        scratch_types=[pltpu.VMEM((NH * NP,), jnp.float32)]
        + chunk_bufs + chunk_bufs
        + [pltpu.VMEM((NW * 16,), jnp.float32)]
        + [pltpu.SemaphoreType.DMA] * 3,
        compiler_params=pltpu.CompilerParams(needs_layout_passes=False),
    )(_segsum_body)
    return kern(ap[0], ap[1], ap[2], ap[3], src, m_part, zeros)


def _out_body(h_ref, s_ref, rsel_ref, msel_ref, out_ref):
    acc = jnp.dot(rsel_ref[...], s_ref[...],
                  preferred_element_type=jnp.float32)
    s_exp = lax.dot_general(
        acc, msel_ref[...],
        (((0,), (0,)), ((), ())),
        preferred_element_type=jnp.float32,
    )
    out_ref[...] = h_ref[...] * (s_exp / (s_exp + jnp.float32(1e-8)))


def _combine(H, s_parts, rsel, msel):
    blk = 2048
    grid = pl.cdiv(N_NODES, blk)
    return pl.pallas_call(
        _out_body,
        grid=(grid,),
        in_specs=[
            pl.BlockSpec((blk, D), lambda i: (i, 0)),
            pl.BlockSpec((NW * NH, blk), lambda i: (0, i)),
            pl.BlockSpec((NH, NW * NH), lambda i: (0, 0)),
            pl.BlockSpec((NH, D), lambda i: (0, 0)),
        ],
        out_specs=pl.BlockSpec((blk, D), lambda i: (i, 0)),
        out_shape=jax.ShapeDtypeStruct((N_NODES, D), jnp.float32),
    )(H, s_parts, rsel, msel)


@jax.jit
def kernel(V, E, edges, W_edge, W_att, b_att):
    V2 = V[0]
    E2 = E[0]
    src = edges[0, :, 0]
    dst = edges[0, :, 1]

    wcat = W_edge.transpose(1, 0, 2).reshape(D, D)
    bdmask = np.zeros((D, NH), np.float32)
    for h in range(NH):
        bdmask[h * OPH:(h + 1) * OPH, h] = 1.0
    wsr = jnp.concatenate(
        [W_att[:, :OPH, 0].reshape(D, 1) * bdmask,
         W_att[:, OPH:2 * OPH, 0].reshape(D, 1) * bdmask], axis=1)
    bvec = jnp.pad(b_att[:, 0], (0, 4)).reshape(1, 8)
    waep = jnp.pad(W_att[:, 2 * OPH:, 0], ((0, 4), (0, 0)))
    msel = np.zeros((NH, D), np.float32)
    for h in range(NH):
        msel[h, h * OPH:(h + 1) * OPH] = 1.0
    msel = jnp.asarray(msel)
    rsel = jnp.asarray(np.tile(np.eye(NH, dtype=np.float32), (1, NW)))
    zeros = jnp.zeros((NH * NP,), jnp.float32)

    H, a_tab = _node_tables(V2, wcat, wsr, bvec)
    e_sc = _edge_scores(E2, waep)

    a_flat = a_tab.reshape(N_NODES * 8)
    outs = _edge_logits(a_flat, e_sc, src, dst)
    ap, m_part = outs[:NH], outs[NH]
    s_parts = _segment_sums(ap, src, m_part, zeros)

    out = _combine(H, s_parts.reshape(NW * NH, NP), rsel, msel)
    return out[None]
